# Initial kernel scaffold; baseline (speedup 1.0000x reference)
import jax
import jax.numpy as jnp
from jax.experimental import pallas as pl


def kernel(x, edge_index, W1, b1, W2, b2):
    raise NotImplementedError("write your pallas kernel here")


# speedup vs baseline: 51.4102x; 51.4102x over previous
import functools

import jax
import jax.numpy as jnp
from jax import lax
from jax.experimental import pallas as pl
from jax.experimental.pallas import tpu as pltpu
from jax.experimental.pallas import tpu_sc as plsc

N = 10000
D = 16
LANES = 16
NC = 2
NS = 16
NW = NC * NS
CHUNK = 128

NP2 = 10240
DEG_SLAB2 = NP2 // NS
DIS_SLAB = NP2 // NW

NT_AGG = 10016
AGG_ZSLAB = NT_AGG // NS
AGG_WSLAB = N // NS

_MESH = plsc.VectorSubcoreMesh(
    core_axis_name="c", subcore_axis_name="s", num_cores=NC, num_subcores=NS
)
_SC_PARAMS = pltpu.CompilerParams(
    use_tc_tiling_on_sc=False, needs_layout_passes=False
)


def _deg_body(cpw2, dst_hbm, dis_hbm, dst_v, ones_v, zer_v, dis_loc, deg_sh,
              s0, s1, si):
    c = lax.axis_index("c")
    s = lax.axis_index("s")

    cp_idx = pltpu.make_async_copy(dst_hbm.at[s], dst_v, si)
    cp_idx.start()

    for i in range(CHUNK // LANES):
        ones_v[pl.ds(i * LANES, LANES)] = jnp.full((LANES,), 1.0, jnp.float32)

    def zfill(i, carry):
        zer_v[pl.ds(i * LANES, LANES)] = jnp.zeros((LANES,), jnp.float32)
        return carry

    lax.fori_loop(0, DEG_SLAB2 // LANES, zfill, 0)
    pltpu.sync_copy(zer_v, deg_sh.at[pl.ds(s * DEG_SLAB2, DEG_SLAB2)])
    cp_idx.wait()
    plsc.subcore_barrier()

    def sc_start(j, sem):
        pltpu.make_async_copy(ones_v, deg_sh.at[dst_v.at[j]], sem).start(add=True)

    def sc_wait(sem):
        pltpu.make_async_copy(ones_v, deg_sh.at[dst_v.at[0]], sem).wait()

    sc_start(0, s0)
    sc_start(1, s1)

    def body(i, carry):
        j = 2 * i + 2
        sc_wait(s0)
        sc_start(j, s0)
        sc_wait(s1)
        sc_start(j + 1, s1)
        return carry

    lax.fori_loop(0, (cpw2 - 2) // 2, body, 0)
    sc_wait(s0)
    sc_wait(s1)
    plsc.subcore_barrier()

    base = (c * NS + s) * DIS_SLAB
    pltpu.sync_copy(deg_sh.at[pl.ds(base, DIS_SLAB)], dis_loc)

    def newton(i, carry):
        d = dis_loc[pl.ds(i * LANES, LANES)] + 1.0
        h = d * 0.5
        ib = plsc.bitcast(d, jnp.int32)
        ib = 0x5F3759DF - lax.shift_right_logical(ib, 1)
        y = plsc.bitcast(ib, jnp.float32)
        y = y * (1.5 - h * y * y)
        y = y * (1.5 - h * y * y)
        y = y * (1.5 - h * y * y)
        dis_loc[pl.ds(i * LANES, LANES)] = y
        return carry

    lax.fori_loop(0, DIS_SLAB // LANES, newton, 0)
    pltpu.sync_copy(dis_loc, dis_hbm.at[pl.ds(base, DIS_SLAB)])


def _sc_deg(dst_deg, cpw2):
    body = functools.partial(_deg_body, cpw2)
    return pl.kernel(
        body,
        out_type=jax.ShapeDtypeStruct((NP2,), jnp.float32),
        mesh=_MESH,
        compiler_params=_SC_PARAMS,
        scratch_types=[
            pltpu.VMEM((cpw2, CHUNK), jnp.int32),
            pltpu.VMEM((CHUNK,), jnp.float32),
            pltpu.VMEM((DEG_SLAB2,), jnp.float32),
            pltpu.VMEM((DIS_SLAB,), jnp.float32),
            pltpu.VMEM_SHARED((NP2,), jnp.float32),
            pltpu.SemaphoreType.DMA,
            pltpu.SemaphoreType.DMA,
            pltpu.SemaphoreType.DMA,
        ],
    )(dst_deg)


def _agg_body(cpw, src_hbm, dst_hbm, t_hbm, out_hbm, src_v, dst_v,
              r0, r1, r2, r3, zer_v, agg_sh,
              sg0, sg1, sg2, sg3, ss0, ss1, ss2, ss3, si0, si1):
    rows = (r0, r1, r2, r3)
    sg = (sg0, sg1, sg2, sg3)
    ss = (ss0, ss1, ss2, ss3)
    c = lax.axis_index("c")
    s = lax.axis_index("s")
    wid = c * NS + s

    cp_src = pltpu.make_async_copy(src_hbm.at[wid], src_v, si0)
    cp_src.start()
    cp_dst = pltpu.make_async_copy(dst_hbm.at[wid], dst_v, si1)
    cp_dst.start()

    def zfill(i, carry):
        zer_v[i] = jnp.zeros((LANES,), jnp.float32)
        return carry

    lax.fori_loop(0, AGG_ZSLAB, zfill, 0)
    pltpu.sync_copy(zer_v, agg_sh.at[pl.ds(s * AGG_ZSLAB, AGG_ZSLAB)])
    cp_src.wait()
    cp_dst.wait()
    plsc.subcore_barrier()

    def g_start(j, b):
        pltpu.make_async_copy(t_hbm.at[src_v.at[j]], rows[b], sg[b]).start()

    def slot(j, b, first):
        pltpu.make_async_copy(t_hbm.at[src_v.at[j]], rows[b], sg[b]).wait()
        pltpu.make_async_copy(rows[b], agg_sh.at[dst_v.at[j]], ss[b]).start(
            add=True)
        b2 = (b + 2) % 4
        if not first:
            pltpu.make_async_copy(rows[b2], agg_sh.at[dst_v.at[j]], ss[b2]).wait()
        g_start(jnp.minimum(j + 2, cpw - 1), b2)

    g_start(0, 0)
    g_start(1, 1)
    slot(0, 0, True)
    slot(1, 1, True)

    def body(k, carry):
        j0 = 2 + 4 * k
        slot(j0, 2, False)
        slot(j0 + 1, 3, False)
        slot(j0 + 2, 0, False)
        slot(j0 + 3, 1, False)
        return carry

    lax.fori_loop(0, (cpw - 4) // 4, body, 0)
    slot(cpw - 2, 2, False)
    slot(cpw - 1, 3, False)
    pltpu.make_async_copy(t_hbm.at[src_v.at[0]], rows[0], sg[0]).wait()
    pltpu.make_async_copy(t_hbm.at[src_v.at[0]], rows[1], sg[1]).wait()
    pltpu.make_async_copy(rows[2], agg_sh.at[dst_v.at[0]], ss[2]).wait()
    pltpu.make_async_copy(rows[3], agg_sh.at[dst_v.at[0]], ss[3]).wait()
    plsc.subcore_barrier()

    pltpu.sync_copy(
        agg_sh.at[pl.ds(s * AGG_WSLAB, AGG_WSLAB)],
        out_hbm.at[c, pl.ds(s * AGG_WSLAB, AGG_WSLAB)],
    )


def _sc_agg(src3, dst3, t, cpw):
    body = functools.partial(_agg_body, cpw)
    return pl.kernel(
        body,
        out_type=jax.ShapeDtypeStruct((NC, N, D), jnp.float32),
        mesh=_MESH,
        compiler_params=_SC_PARAMS,
        scratch_types=[
            pltpu.VMEM((cpw, CHUNK), jnp.int32),
            pltpu.VMEM((cpw, CHUNK), jnp.int32),
            pltpu.VMEM((CHUNK, D), jnp.float32),
            pltpu.VMEM((CHUNK, D), jnp.float32),
            pltpu.VMEM((CHUNK, D), jnp.float32),
            pltpu.VMEM((CHUNK, D), jnp.float32),
            pltpu.VMEM((AGG_ZSLAB, D), jnp.float32),
            pltpu.VMEM_SHARED((NT_AGG, D), jnp.float32),
            pltpu.SemaphoreType.DMA,
            pltpu.SemaphoreType.DMA,
            pltpu.SemaphoreType.DMA,
            pltpu.SemaphoreType.DMA,
            pltpu.SemaphoreType.DMA,
            pltpu.SemaphoreType.DMA,
            pltpu.SemaphoreType.DMA,
            pltpu.SemaphoreType.DMA,
            pltpu.SemaphoreType.DMA,
            pltpu.SemaphoreType.DMA,
        ],
    )(src3, dst3, t)


def _t1_body(x_ref, w1_ref, dis_ref, t1_ref):
    xw = jnp.dot(x_ref[...], w1_ref[...], preferred_element_type=jnp.float32)
    t1_ref[...] = xw * dis_ref[...]


def _mid_body(t1_ref, p_ref, dis_ref, b1_ref, w2_ref, t2_ref):
    tot = t1_ref[...] + p_ref[0] + p_ref[1]
    h = jnp.maximum(tot * dis_ref[...] + b1_ref[...], 0.0)
    hw = jnp.dot(h, w2_ref[...], preferred_element_type=jnp.float32)
    t2_ref[...] = hw * dis_ref[...]


def _out_body(t2_ref, p_ref, dis_ref, b2_ref, o_ref):
    tot = t2_ref[...] + p_ref[0] + p_ref[1]
    o_ref[...] = tot * dis_ref[...] + b2_ref[...]


def _tc(body, out_shape, *args):
    return pl.pallas_call(
        body, out_shape=jax.ShapeDtypeStruct(out_shape, jnp.float32)
    )(*args)


def kernel(x, edge_index, W1, b1, W2, b2):
    E = edge_index.shape[1]
    cpw = -(-E // (NW * CHUNK))
    while cpw % 4 != 0:
        cpw += 1
    epad = NW * cpw * CHUNK
    pad = epad - E
    pad_i = jnp.arange(pad, dtype=jnp.int32)
    src_flat = jnp.concatenate([edge_index[0], pad_i % N])
    dst_flat = jnp.concatenate([edge_index[1], N + pad_i % 16])
    src3 = src_flat.reshape(NW, cpw, CHUNK)
    dst3 = dst_flat.reshape(NW, cpw, CHUNK)
    dst_deg = dst_flat.reshape(NS, 2 * cpw, CHUNK)

    dis_full = _sc_deg(dst_deg, 2 * cpw)
    dis_col = dis_full[:N].reshape(N, 1)

    t1 = _tc(_t1_body, (N, D), x, W1, dis_col)
    a1 = _sc_agg(src3, dst3, t1, cpw)
    t2 = _tc(_mid_body, (N, D), t1, a1, dis_col,
             b1.reshape(1, D), W2)
    a2 = _sc_agg(src3, dst3, t2, cpw)
    out = _tc(_out_body, (N, D), t2, a2, dis_col, b2.reshape(1, D))
    return out


# speedup vs baseline: 60.0754x; 1.1686x over previous
import functools

import jax
import jax.numpy as jnp
from jax import lax
from jax.experimental import pallas as pl
from jax.experimental.pallas import tpu as pltpu
from jax.experimental.pallas import tpu_sc as plsc

N = 10000
D = 16
LANES = 16
NC = 2
NS = 16
NW = NC * NS
CHUNK = 128

NP2 = 10240
DEG_SLAB2 = NP2 // NS
DIS_SLAB = NP2 // NW

NT_AGG = 10016
AGG_ZSLAB = NT_AGG // NS
AGG_WSLAB = N // NS

_MESH = plsc.VectorSubcoreMesh(
    core_axis_name="c", subcore_axis_name="s", num_cores=NC, num_subcores=NS
)
_SC_PARAMS = pltpu.CompilerParams(
    use_tc_tiling_on_sc=False, needs_layout_passes=False
)


def _deg_body(cpw2, dst_hbm, dis_hbm, dst_v, ones_v, zer_v, dis_loc, deg_sh,
              s0, s1, si):
    c = lax.axis_index("c")
    s = lax.axis_index("s")

    cp_idx = pltpu.make_async_copy(dst_hbm.at[s], dst_v, si)
    cp_idx.start()

    for i in range(CHUNK // LANES):
        ones_v[pl.ds(i * LANES, LANES)] = jnp.full((LANES,), 1.0, jnp.float32)

    def zfill(i, carry):
        zer_v[pl.ds(i * LANES, LANES)] = jnp.zeros((LANES,), jnp.float32)
        return carry

    lax.fori_loop(0, DEG_SLAB2 // LANES, zfill, 0)
    pltpu.sync_copy(zer_v, deg_sh.at[pl.ds(s * DEG_SLAB2, DEG_SLAB2)])
    cp_idx.wait()
    plsc.subcore_barrier()

    def sc_start(j, sem):
        pltpu.make_async_copy(ones_v, deg_sh.at[dst_v.at[j]], sem).start(add=True)

    def sc_wait(sem):
        pltpu.make_async_copy(ones_v, deg_sh.at[dst_v.at[0]], sem).wait()

    sc_start(0, s0)
    sc_start(1, s1)

    def body(i, carry):
        j = 2 * i + 2
        sc_wait(s0)
        sc_start(j, s0)
        sc_wait(s1)
        sc_start(j + 1, s1)
        return carry

    lax.fori_loop(0, (cpw2 - 2) // 2, body, 0)
    sc_wait(s0)
    sc_wait(s1)
    plsc.subcore_barrier()

    base = (c * NS + s) * DIS_SLAB
    pltpu.sync_copy(deg_sh.at[pl.ds(base, DIS_SLAB)], dis_loc)

    def newton(i, carry):
        d = dis_loc[pl.ds(i * LANES, LANES)] + 1.0
        h = d * 0.5
        ib = plsc.bitcast(d, jnp.int32)
        ib = 0x5F3759DF - lax.shift_right_logical(ib, 1)
        y = plsc.bitcast(ib, jnp.float32)
        y = y * (1.5 - h * y * y)
        y = y * (1.5 - h * y * y)
        y = y * (1.5 - h * y * y)
        dis_loc[pl.ds(i * LANES, LANES)] = y
        return carry

    lax.fori_loop(0, DIS_SLAB // LANES, newton, 0)
    pltpu.sync_copy(dis_loc, dis_hbm.at[pl.ds(base, DIS_SLAB)])


def _sc_deg(dst_deg, cpw2):
    body = functools.partial(_deg_body, cpw2)
    return pl.kernel(
        body,
        out_type=jax.ShapeDtypeStruct((NP2,), jnp.float32),
        mesh=_MESH,
        compiler_params=_SC_PARAMS,
        scratch_types=[
            pltpu.VMEM((cpw2, CHUNK), jnp.int32),
            pltpu.VMEM((CHUNK,), jnp.float32),
            pltpu.VMEM((DEG_SLAB2,), jnp.float32),
            pltpu.VMEM((DIS_SLAB,), jnp.float32),
            pltpu.VMEM_SHARED((NP2,), jnp.float32),
            pltpu.SemaphoreType.DMA,
            pltpu.SemaphoreType.DMA,
            pltpu.SemaphoreType.DMA,
        ],
    )(dst_deg)


NBUF = 8
PF = NBUF // 2


def _agg_body(cpw, src_hbm, dst_hbm, t_hbm, out_hbm, src_v, dst_v,
              *scr):
    rows = scr[:NBUF]
    zer_v = scr[NBUF]
    agg_sh = scr[NBUF + 1]
    sg = scr[NBUF + 2:2 * NBUF + 2]
    ss = scr[2 * NBUF + 2:3 * NBUF + 2]
    si0, si1 = scr[3 * NBUF + 2:]
    c = lax.axis_index("c")
    s = lax.axis_index("s")
    wid = c * NS + s

    cp_src = pltpu.make_async_copy(src_hbm.at[wid], src_v, si0)
    cp_src.start()
    cp_dst = pltpu.make_async_copy(dst_hbm.at[wid], dst_v, si1)
    cp_dst.start()

    def zfill(i, carry):
        zer_v[i] = jnp.zeros((LANES,), jnp.float32)
        return carry

    lax.fori_loop(0, AGG_ZSLAB, zfill, 0)
    pltpu.sync_copy(zer_v, agg_sh.at[pl.ds(s * AGG_ZSLAB, AGG_ZSLAB)])
    cp_src.wait()
    cp_dst.wait()
    plsc.subcore_barrier()

    def g_start(j, b):
        pltpu.make_async_copy(t_hbm.at[src_v.at[j]], rows[b], sg[b]).start()

    def slot(j, b, first):
        pltpu.make_async_copy(t_hbm.at[src_v.at[j]], rows[b], sg[b]).wait()
        pltpu.make_async_copy(rows[b], agg_sh.at[dst_v.at[j]], ss[b]).start(
            add=True)
        b2 = (b + PF) % NBUF
        if not first:
            pltpu.make_async_copy(rows[b2], agg_sh.at[dst_v.at[j]], ss[b2]).wait()
        g_start(jnp.minimum(j + PF, cpw - 1), b2)

    for b in range(PF):
        g_start(b, b)
    for b in range(PF):
        slot(b, b, True)

    def body(k, carry):
        j0 = PF + NBUF * k
        for i in range(NBUF):
            slot(j0 + i, (PF + i) % NBUF, False)
        return carry

    lax.fori_loop(0, (cpw - 2 * PF) // NBUF, body, 0)
    for i in range(PF):
        slot(cpw - PF + i, (cpw - PF + i) % NBUF, False)
    for b in range(PF):
        pltpu.make_async_copy(t_hbm.at[src_v.at[0]], rows[b], sg[b]).wait()
    for i in range(PF):
        b = (cpw - PF + i) % NBUF
        pltpu.make_async_copy(rows[b], agg_sh.at[dst_v.at[0]], ss[b]).wait()
    plsc.subcore_barrier()

    pltpu.sync_copy(
        agg_sh.at[pl.ds(s * AGG_WSLAB, AGG_WSLAB)],
        out_hbm.at[c, pl.ds(s * AGG_WSLAB, AGG_WSLAB)],
    )


def _sc_agg(src3, dst3, t, cpw):
    body = functools.partial(_agg_body, cpw)
    return pl.kernel(
        body,
        out_type=jax.ShapeDtypeStruct((NC, N, D), jnp.float32),
        mesh=_MESH,
        compiler_params=_SC_PARAMS,
        scratch_types=(
            [pltpu.VMEM((cpw, CHUNK), jnp.int32)] * 2
            + [pltpu.VMEM((CHUNK, D), jnp.float32)] * NBUF
            + [pltpu.VMEM((AGG_ZSLAB, D), jnp.float32),
               pltpu.VMEM_SHARED((NT_AGG, D), jnp.float32)]
            + [pltpu.SemaphoreType.DMA] * (2 * NBUF + 2)
        ),
    )(src3, dst3, t)


def _t1_body(x_ref, w1_ref, dis_ref, t1_ref):
    xw = jnp.dot(x_ref[...], w1_ref[...], preferred_element_type=jnp.float32)
    t1_ref[...] = xw * dis_ref[...]


def _mid_body(t1_ref, p_ref, dis_ref, b1_ref, w2_ref, t2_ref):
    tot = t1_ref[...] + p_ref[0] + p_ref[1]
    h = jnp.maximum(tot * dis_ref[...] + b1_ref[...], 0.0)
    hw = jnp.dot(h, w2_ref[...], preferred_element_type=jnp.float32)
    t2_ref[...] = hw * dis_ref[...]


def _out_body(t2_ref, p_ref, dis_ref, b2_ref, o_ref):
    tot = t2_ref[...] + p_ref[0] + p_ref[1]
    o_ref[...] = tot * dis_ref[...] + b2_ref[...]


def _tc(body, out_shape, *args):
    return pl.pallas_call(
        body, out_shape=jax.ShapeDtypeStruct(out_shape, jnp.float32)
    )(*args)


def kernel(x, edge_index, W1, b1, W2, b2):
    E = edge_index.shape[1]
    cpw = -(-E // (NW * CHUNK))
    while cpw % NBUF != 0:
        cpw += 1
    epad = NW * cpw * CHUNK
    pad = epad - E
    pad_i = jnp.arange(pad, dtype=jnp.int32)
    src_flat = jnp.concatenate([edge_index[0], pad_i % N])
    dst_flat = jnp.concatenate([edge_index[1], N + pad_i % 16])
    src3 = src_flat.reshape(NW, cpw, CHUNK)
    dst3 = dst_flat.reshape(NW, cpw, CHUNK)
    dst_deg = dst_flat.reshape(NS, 2 * cpw, CHUNK)

    dis_full = _sc_deg(dst_deg, 2 * cpw)
    dis_col = dis_full[:N].reshape(N, 1)

    t1 = _tc(_t1_body, (N, D), x, W1, dis_col)
    a1 = _sc_agg(src3, dst3, t1, cpw)
    t2 = _tc(_mid_body, (N, D), t1, a1, dis_col,
             b1.reshape(1, D), W2)
    a2 = _sc_agg(src3, dst3, t2, cpw)
    out = _tc(_out_body, (N, D), t2, a2, dis_col, b2.reshape(1, D))
    return out


# speedup vs baseline: 61.0751x; 1.0166x over previous
import functools

import jax
import jax.numpy as jnp
from jax import lax
from jax.experimental import pallas as pl
from jax.experimental.pallas import tpu as pltpu
from jax.experimental.pallas import tpu_sc as plsc

N = 10000
D = 16
LANES = 16
NC = 2
NS = 16
NW = NC * NS
CHUNK = 128

NP2 = 10240
DEG_SLAB2 = NP2 // NS
DIS_SLAB = NP2 // NW

NT_AGG = 10016
AGG_ZSLAB = NT_AGG // NS
AGG_WSLAB = N // NS

_MESH = plsc.VectorSubcoreMesh(
    core_axis_name="c", subcore_axis_name="s", num_cores=NC, num_subcores=NS
)
_SC_PARAMS = pltpu.CompilerParams(
    use_tc_tiling_on_sc=False, needs_layout_passes=False
)


def _deg_body(cpw2, dst_hbm, dis_hbm, dst_v, ones_v, zer_v, dis_loc, deg_sh,
              s0, s1, s2, s3, si):
    c = lax.axis_index("c")
    s = lax.axis_index("s")

    cp_idx = pltpu.make_async_copy(dst_hbm.at[s], dst_v, si)
    cp_idx.start()

    for i in range(CHUNK // LANES):
        ones_v[pl.ds(i * LANES, LANES)] = jnp.full((LANES,), 1.0, jnp.float32)

    def zfill(i, carry):
        zer_v[pl.ds(i * LANES, LANES)] = jnp.zeros((LANES,), jnp.float32)
        return carry

    lax.fori_loop(0, DEG_SLAB2 // LANES, zfill, 0)
    pltpu.sync_copy(zer_v, deg_sh.at[pl.ds(s * DEG_SLAB2, DEG_SLAB2)])
    cp_idx.wait()
    plsc.subcore_barrier()

    def sc_start(j, sem):
        pltpu.make_async_copy(ones_v, deg_sh.at[dst_v.at[j]], sem).start(add=True)

    def sc_wait(sem):
        pltpu.make_async_copy(ones_v, deg_sh.at[dst_v.at[0]], sem).wait()

    dsems = (s0, s1, s2, s3)
    for b in range(4):
        sc_start(b, dsems[b])

    def body(i, carry):
        j = 4 * i + 4
        for b in range(4):
            sc_wait(dsems[b])
            sc_start(j + b, dsems[b])
        return carry

    lax.fori_loop(0, (cpw2 - 4) // 4, body, 0)
    for b in range(4):
        sc_wait(dsems[b])
    plsc.subcore_barrier()

    base = (c * NS + s) * DIS_SLAB
    pltpu.sync_copy(deg_sh.at[pl.ds(base, DIS_SLAB)], dis_loc)

    def newton(i, carry):
        d = dis_loc[pl.ds(i * LANES, LANES)] + 1.0
        h = d * 0.5
        ib = plsc.bitcast(d, jnp.int32)
        ib = 0x5F3759DF - lax.shift_right_logical(ib, 1)
        y = plsc.bitcast(ib, jnp.float32)
        y = y * (1.5 - h * y * y)
        y = y * (1.5 - h * y * y)
        y = y * (1.5 - h * y * y)
        dis_loc[pl.ds(i * LANES, LANES)] = y
        return carry

    lax.fori_loop(0, DIS_SLAB // LANES, newton, 0)
    pltpu.sync_copy(dis_loc, dis_hbm.at[pl.ds(base, DIS_SLAB)])


def _sc_deg(dst_deg, cpw2):
    body = functools.partial(_deg_body, cpw2)
    return pl.kernel(
        body,
        out_type=jax.ShapeDtypeStruct((NP2,), jnp.float32),
        mesh=_MESH,
        compiler_params=_SC_PARAMS,
        scratch_types=[
            pltpu.VMEM((cpw2, CHUNK), jnp.int32),
            pltpu.VMEM((CHUNK,), jnp.float32),
            pltpu.VMEM((DEG_SLAB2,), jnp.float32),
            pltpu.VMEM((DIS_SLAB,), jnp.float32),
            pltpu.VMEM_SHARED((NP2,), jnp.float32),
            pltpu.SemaphoreType.DMA,
            pltpu.SemaphoreType.DMA,
            pltpu.SemaphoreType.DMA,
            pltpu.SemaphoreType.DMA,
            pltpu.SemaphoreType.DMA,
        ],
    )(dst_deg)


NBUF = 8
PF = NBUF // 2


def _agg_body(cpw, src_hbm, dst_hbm, t_hbm, out_hbm, src_v, dst_v,
              *scr):
    rows = scr[:NBUF]
    zer_v = scr[NBUF]
    agg_sh = scr[NBUF + 1]
    sg = scr[NBUF + 2:2 * NBUF + 2]
    ss = scr[2 * NBUF + 2:3 * NBUF + 2]
    si0, si1 = scr[3 * NBUF + 2:]
    c = lax.axis_index("c")
    s = lax.axis_index("s")
    wid = c * NS + s

    cp_src = pltpu.make_async_copy(src_hbm.at[wid], src_v, si0)
    cp_src.start()
    cp_dst = pltpu.make_async_copy(dst_hbm.at[wid], dst_v, si1)
    cp_dst.start()

    def zfill(i, carry):
        zer_v[i] = jnp.zeros((LANES,), jnp.float32)
        return carry

    lax.fori_loop(0, AGG_ZSLAB, zfill, 0)
    pltpu.sync_copy(zer_v, agg_sh.at[pl.ds(s * AGG_ZSLAB, AGG_ZSLAB)])
    cp_src.wait()
    cp_dst.wait()
    plsc.subcore_barrier()

    def g_start(j, b):
        pltpu.make_async_copy(t_hbm.at[src_v.at[j]], rows[b], sg[b]).start()

    def slot(j, b, first):
        pltpu.make_async_copy(t_hbm.at[src_v.at[j]], rows[b], sg[b]).wait()
        pltpu.make_async_copy(rows[b], agg_sh.at[dst_v.at[j]], ss[b]).start(
            add=True)
        b2 = (b + PF) % NBUF
        if not first:
            pltpu.make_async_copy(rows[b2], agg_sh.at[dst_v.at[j]], ss[b2]).wait()
        g_start(jnp.minimum(j + PF, cpw - 1), b2)

    for b in range(PF):
        g_start(b, b)
    for b in range(PF):
        slot(b, b, True)

    def body(k, carry):
        j0 = PF + NBUF * k
        for i in range(NBUF):
            slot(j0 + i, (PF + i) % NBUF, False)
        return carry

    lax.fori_loop(0, (cpw - 2 * PF) // NBUF, body, 0)
    for i in range(PF):
        slot(cpw - PF + i, (cpw - PF + i) % NBUF, False)
    for b in range(PF):
        pltpu.make_async_copy(t_hbm.at[src_v.at[0]], rows[b], sg[b]).wait()
    for i in range(PF):
        b = (cpw - PF + i) % NBUF
        pltpu.make_async_copy(rows[b], agg_sh.at[dst_v.at[0]], ss[b]).wait()
    plsc.subcore_barrier()

    pltpu.sync_copy(
        agg_sh.at[pl.ds(s * AGG_WSLAB, AGG_WSLAB)],
        out_hbm.at[c, pl.ds(s * AGG_WSLAB, AGG_WSLAB)],
    )


def _sc_agg(src3, dst3, t, cpw):
    body = functools.partial(_agg_body, cpw)
    return pl.kernel(
        body,
        out_type=jax.ShapeDtypeStruct((NC, N, D), jnp.float32),
        mesh=_MESH,
        compiler_params=_SC_PARAMS,
        scratch_types=(
            [pltpu.VMEM((cpw, CHUNK), jnp.int32)] * 2
            + [pltpu.VMEM((CHUNK, D), jnp.float32)] * NBUF
            + [pltpu.VMEM((AGG_ZSLAB, D), jnp.float32),
               pltpu.VMEM_SHARED((NT_AGG, D), jnp.float32)]
            + [pltpu.SemaphoreType.DMA] * (2 * NBUF + 2)
        ),
    )(src3, dst3, t)


def _t1_body(x_ref, w1_ref, dis_ref, t1_ref):
    xw = jnp.dot(x_ref[...], w1_ref[...], preferred_element_type=jnp.float32)
    t1_ref[...] = xw * dis_ref[...]


def _mid_body(t1_ref, p_ref, dis_ref, b1_ref, w2_ref, t2_ref):
    tot = t1_ref[...] + p_ref[0] + p_ref[1]
    h = jnp.maximum(tot * dis_ref[...] + b1_ref[...], 0.0)
    hw = jnp.dot(h, w2_ref[...], preferred_element_type=jnp.float32)
    t2_ref[...] = hw * dis_ref[...]


def _out_body(t2_ref, p_ref, dis_ref, b2_ref, o_ref):
    tot = t2_ref[...] + p_ref[0] + p_ref[1]
    o_ref[...] = tot * dis_ref[...] + b2_ref[...]


def _tc(body, out_shape, *args):
    return pl.pallas_call(
        body, out_shape=jax.ShapeDtypeStruct(out_shape, jnp.float32)
    )(*args)


def kernel(x, edge_index, W1, b1, W2, b2):
    E = edge_index.shape[1]
    cpw = -(-E // (NW * CHUNK))
    while cpw % NBUF != 0:
        cpw += 1
    epad = NW * cpw * CHUNK
    pad = epad - E
    pad_i = jnp.arange(pad, dtype=jnp.int32)
    src_flat = jnp.concatenate([edge_index[0], pad_i % N])
    dst_flat = jnp.concatenate([edge_index[1], N + pad_i % 16])
    src3 = src_flat.reshape(NW, cpw, CHUNK)
    dst3 = dst_flat.reshape(NW, cpw, CHUNK)
    dst_deg = dst_flat.reshape(NS, 2 * cpw, CHUNK)

    dis_full = _sc_deg(dst_deg, 2 * cpw)
    dis_col = dis_full[:N].reshape(N, 1)

    t1 = _tc(_t1_body, (N, D), x, W1, dis_col)
    a1 = _sc_agg(src3, dst3, t1, cpw)
    t2 = _tc(_mid_body, (N, D), t1, a1, dis_col,
             b1.reshape(1, D), W2)
    a2 = _sc_agg(src3, dst3, t2, cpw)
    out = _tc(_out_body, (N, D), t2, a2, dis_col, b2.reshape(1, D))
    return out
